# Initial kernel scaffold; baseline (speedup 1.0000x reference)
import functools

import jax
import jax.numpy as jnp
from jax import lax
from jax.experimental import pallas as pl
from jax.experimental.pallas import tpu as pltpu
from jax.experimental.pallas import tpu_sc as plsc

F32 = jnp.float32

NC = 2
NS = 16
NW = NC * NS
LANE = 16

H = 64
UW = 80
EC = 128


def _qkvs_body(x_ref, w_ref, b_ref, oq_ref, ok_ref, ov_ref, os_ref):
    r = jnp.dot(x_ref[...], w_ref[...], preferred_element_type=F32) + b_ref[...]
    oq_ref[...] = r[:, 0:64]
    ok_ref[...] = r[:, 64:128]
    ov_ref[...] = r[:, 128:192]
    os_ref[...] = r[:, 192:256]


def _qkvs(x, wcat, bcat):
    n, d = x.shape
    rb = 2000
    grid = (n // rb,)
    out = jax.ShapeDtypeStruct((n, H), F32)
    return pl.pallas_call(
        _qkvs_body,
        grid=grid,
        in_specs=[
            pl.BlockSpec((rb, d), lambda i: (i, 0)),
            pl.BlockSpec((d, 4 * H), lambda i: (0, 0)),
            pl.BlockSpec((1, 4 * H), lambda i: (0, 0)),
        ],
        out_specs=[pl.BlockSpec((rb, H), lambda i: (i, 0))] * 4,
        out_shape=[out, out, out, out],
    )(x, wcat, bcat)


def _edge_body(q_hbm, k_hbm, v_hbm, src_hbm, dst_hbm, u_hbm,
               src_v, dst_v, q_r, k_r, v_r, o_r, w_r, u_sh,
               sem1, sem2, sem3):
    n = q_hbm.shape[0]
    e = src_hbm.shape[0]
    cid = lax.axis_index("c")
    sid = lax.axis_index("s")
    wid = sid * NC + cid

    rows_per_tile = n // NS
    zrows = 125

    @pl.loop(0, zrows)
    def _(i):
        @pl.loop(0, UW, step=LANE)
        def _(j):
            o_r[i, pl.ds(j, LANE)] = jnp.zeros((LANE,), F32)

    @pl.loop(0, rows_per_tile // zrows)
    def _(j):
        pltpu.sync_copy(o_r.at[pl.ds(0, zrows)],
                        u_sh.at[pl.ds(sid * rows_per_tile + j * zrows, zrows)])

    plsc.subcore_barrier()

    lane_iota = lax.iota(jnp.int32, LANE)
    nchunks = e // EC

    @pl.loop(wid, nchunks, step=NW)
    def _(g):
        base = g * EC
        pltpu.sync_copy(src_hbm.at[pl.ds(base, EC)], src_v.at[0])
        pltpu.sync_copy(dst_hbm.at[pl.ds(base, EC)], dst_v.at[0])
        cp1 = pltpu.async_copy(q_hbm.at[dst_v.at[0]], q_r, sem1)
        cp2 = pltpu.async_copy(k_hbm.at[src_v.at[0]], k_r, sem2)
        cp3 = pltpu.async_copy(v_hbm.at[src_v.at[0]], v_r, sem3)
        cp1.wait()
        cp2.wait()

        @pl.loop(0, EC)
        def _(i):
            acc = q_r[i, pl.ds(0, LANE)] * k_r[i, pl.ds(0, LANE)]
            acc = acc + q_r[i, pl.ds(16, LANE)] * k_r[i, pl.ds(16, LANE)]
            acc = acc + q_r[i, pl.ds(32, LANE)] * k_r[i, pl.ds(32, LANE)]
            acc = acc + q_r[i, pl.ds(48, LANE)] * k_r[i, pl.ds(48, LANE)]
            w_r[i] = jnp.sum(acc) * 0.125

        @pl.loop(0, EC, step=LANE)
        def _(j):
            w_r[pl.ds(j, LANE)] = jnp.exp(w_r[pl.ds(j, LANE)])

        cp3.wait()

        @pl.loop(0, EC)
        def _(i):
            w = w_r[i]
            o_r[i, pl.ds(0, LANE)] = v_r[i, pl.ds(0, LANE)] * w
            o_r[i, pl.ds(16, LANE)] = v_r[i, pl.ds(16, LANE)] * w
            o_r[i, pl.ds(32, LANE)] = v_r[i, pl.ds(32, LANE)] * w
            o_r[i, pl.ds(48, LANE)] = v_r[i, pl.ds(48, LANE)] * w
            o_r[i, pl.ds(64, LANE)] = jnp.where(lane_iota == 0, w, 0.0)

        pltpu.sync_copy(o_r, u_sh.at[dst_v.at[0]], add=True)

    plsc.subcore_barrier()

    @pl.loop(0, rows_per_tile // zrows)
    def _(j):
        r0 = sid * rows_per_tile + j * zrows
        pltpu.sync_copy(u_sh.at[pl.ds(r0, zrows)], u_hbm.at[cid, pl.ds(r0, zrows)])


def _edge_pass(q, k, v, src, dst):
    n = q.shape[0]
    mesh = plsc.VectorSubcoreMesh(core_axis_name="c", subcore_axis_name="s")
    f = pl.kernel(
        _edge_body,
        out_type=jax.ShapeDtypeStruct((NC, n, UW), F32),
        mesh=mesh,
        scratch_types=[
            pltpu.VMEM((1, EC), jnp.int32),
            pltpu.VMEM((1, EC), jnp.int32),
            pltpu.VMEM((EC, H), F32),
            pltpu.VMEM((EC, H), F32),
            pltpu.VMEM((EC, H), F32),
            pltpu.VMEM((EC, UW), F32),
            pltpu.VMEM((EC,), F32),
            pltpu.VMEM_SHARED((n, UW), F32),
            pltpu.SemaphoreType.DMA,
            pltpu.SemaphoreType.DMA,
            pltpu.SemaphoreType.DMA,
        ],
    )
    return f(q, k, v, src, dst)


def _post_body(u_ref, s_ref, o_ref):
    u = u_ref[0] + u_ref[1]
    agg = u[:, 0:H]
    den = u[:, H:H + 1]
    o_ref[...] = jnp.maximum(agg / (den + 1e-16) + s_ref[...], 0.0)


def _post(u, skip):
    n = skip.shape[0]
    rb = 2000
    return pl.pallas_call(
        _post_body,
        grid=(n // rb,),
        in_specs=[
            pl.BlockSpec((NC, rb, UW), lambda i: (0, i, 0)),
            pl.BlockSpec((rb, H), lambda i: (i, 0)),
        ],
        out_specs=pl.BlockSpec((rb, H), lambda i: (i, 0)),
        out_shape=jax.ShapeDtypeStruct((n, H), F32),
    )(u, skip)


def _pool_body(h_ref, b_ref, w1_ref, b1_ref, w2_ref, b2_ref, o_ref):
    g = o_ref.shape[0]
    n = h_ref.shape[0]
    iota = lax.broadcasted_iota(jnp.int32, (g, n), 0)
    onehot_t = (iota == b_ref[...]).astype(F32)
    sums = jnp.dot(onehot_t, h_ref[...], preferred_element_type=F32)
    cnt = jnp.sum(onehot_t, axis=1, keepdims=True)
    pooled = sums / jnp.maximum(cnt, 1.0)
    t = jnp.dot(pooled, w1_ref[...], preferred_element_type=F32) + b1_ref[...]
    o_ref[...] = jnp.dot(t, w2_ref[...], preferred_element_type=F32) + b2_ref[...]


def _pool(h, batch2d, g, w1, b1, w2, b2):
    return pl.pallas_call(
        _pool_body,
        out_shape=jax.ShapeDtypeStruct((g, 1), F32),
    )(h, batch2d, w1, b1, w2, b2)


def kernel(x, edge_index, batch,
           Wq1, bq1, Wk1, bk1, Wv1, bv1, Ws1, bs1,
           Wq2, bq2, Wk2, bk2, Wv2, bv2, Ws2, bs2,
           Wq3, bq3, Wk3, bk3, Wv3, bv3, Ws3, bs3,
           W_l1, b_l1, W_l2, b_l2):
    x = x.astype(F32)
    src = edge_index[0]
    dst = edge_index[1]
    layers = [
        (Wq1, bq1, Wk1, bk1, Wv1, bv1, Ws1, bs1),
        (Wq2, bq2, Wk2, bk2, Wv2, bv2, Ws2, bs2),
        (Wq3, bq3, Wk3, bk3, Wv3, bv3, Ws3, bs3),
    ]
    h = x
    for (Wq, bq, Wk, bk, Wv, bv, Ws, bs) in layers:
        wcat = jnp.concatenate([Wq, Wk, Wv, Ws], axis=1)
        bcat = jnp.concatenate([bq, bk, bv, bs]).reshape(1, -1)
        q, k, v, skip = _qkvs(h, wcat, bcat)
        u = _edge_pass(q, k, v, src, dst)
        h = _post(u, skip)
    batch2d = batch.reshape(1, -1)
    return _pool(h, batch2d, 64,
                 W_l1, b_l1.reshape(1, -1), W_l2, b_l2.reshape(1, 1))


# speedup vs baseline: 14.3125x; 14.3125x over previous
import dataclasses
import functools

import jax
import jax.numpy as jnp
from jax import lax
from jax.experimental import pallas as pl
from jax.experimental.pallas import tpu as pltpu
from jax.experimental.pallas import tpu_sc as plsc

F32 = jnp.float32

NC = 2
NS = 16
NW = NC * NS
LANE = 16

H = 64
UW = 128
EC = 128


def _qkvs_body(x_ref, w_ref, b_ref, oqs_ref, okv_ref):
    r = jnp.dot(x_ref[...], w_ref[...], preferred_element_type=F32) + b_ref[...]
    oqs_ref[...] = r[:, 0:128]
    okv_ref[...] = r[:, 128:256]


def _qkvs(x, wcat, bcat):
    n, d = x.shape
    rb = 2000
    grid = (n // rb,)
    out = jax.ShapeDtypeStruct((n, 2 * H), F32)
    return pl.pallas_call(
        _qkvs_body,
        grid=grid,
        in_specs=[
            pl.BlockSpec((rb, d), lambda i: (i, 0)),
            pl.BlockSpec((d, 4 * H), lambda i: (0, 0)),
            pl.BlockSpec((1, 4 * H), lambda i: (0, 0)),
        ],
        out_specs=[pl.BlockSpec((rb, 2 * H), lambda i: (i, 0))] * 2,
        out_shape=[out, out],
    )(x, wcat, bcat)


def _edge_body(qs_hbm, kv_hbm, src_hbm, dst_hbm, u_hbm,
               src_v, dst_v, qs_r, kv_r, o_r, u_sh,
               sem1, sem2):
    n = qs_hbm.shape[0]
    e = src_hbm.shape[0]
    cid = lax.axis_index("c")
    sid = lax.axis_index("s")
    wid = sid * NC + cid

    zrows = 80
    nzch = n // zrows

    @pl.loop(0, EC)
    def _(i):
        @pl.loop(0, UW, step=LANE)
        def _(j):
            o_r[i, pl.ds(j, LANE)] = jnp.zeros((LANE,), F32)

    @pl.loop(sid, nzch, step=NS)
    def _(c):
        pltpu.sync_copy(o_r.at[pl.ds(0, zrows)],
                        u_sh.at[pl.ds(c * zrows, zrows)])

    plsc.subcore_barrier()

    lane_iota = lax.iota(jnp.int32, LANE)
    nchunks = e // EC

    @pl.loop(wid, nchunks, step=NW)
    def _(g):
        base = g * EC
        pltpu.sync_copy(src_hbm.at[pl.ds(base, EC)], src_v.at[0])
        pltpu.sync_copy(dst_hbm.at[pl.ds(base, EC)], dst_v.at[0])
        cp1 = pltpu.async_copy(qs_hbm.at[dst_v.at[0]], qs_r, sem1)
        cp2 = pltpu.async_copy(kv_hbm.at[src_v.at[0]], kv_r, sem2)
        cp1.wait()
        cp2.wait()

        @pl.loop(0, EC, step=LANE)
        def _(j):
            vec = jnp.zeros((LANE,), F32)
            for t in range(LANE):
                i = j + t
                acc = qs_r[i, pl.ds(0, LANE)] * kv_r[i, pl.ds(0, LANE)]
                acc = acc + qs_r[i, pl.ds(16, LANE)] * kv_r[i, pl.ds(16, LANE)]
                acc = acc + qs_r[i, pl.ds(32, LANE)] * kv_r[i, pl.ds(32, LANE)]
                acc = acc + qs_r[i, pl.ds(48, LANE)] * kv_r[i, pl.ds(48, LANE)]
                s = jnp.sum(acc) * 0.125
                vec = vec + jnp.where(lane_iota == t, s, 0.0)
            wv = jnp.exp(vec)
            for t in range(LANE):
                i = j + t
                w = wv[t]
                o_r[i, pl.ds(0, LANE)] = kv_r[i, pl.ds(64, LANE)] * w
                o_r[i, pl.ds(16, LANE)] = kv_r[i, pl.ds(80, LANE)] * w
                o_r[i, pl.ds(32, LANE)] = kv_r[i, pl.ds(96, LANE)] * w
                o_r[i, pl.ds(48, LANE)] = kv_r[i, pl.ds(112, LANE)] * w
                o_r[i, pl.ds(64, LANE)] = jnp.where(lane_iota == 0, w, 0.0)

        pltpu.sync_copy(o_r, u_sh.at[dst_v.at[0]], add=True)

    plsc.subcore_barrier()

    @pl.loop(sid, nzch, step=NS)
    def _(c):
        r0 = c * zrows
        pltpu.sync_copy(u_sh.at[pl.ds(r0, zrows)], u_hbm.at[cid, pl.ds(r0, zrows)])


def _edge_pass(qs, kv, src, dst):
    n = qs.shape[0]
    mesh = plsc.VectorSubcoreMesh(core_axis_name="c", subcore_axis_name="s")
    cp = pltpu.CompilerParams()
    if "needs_layout_passes" in pltpu.CompilerParams.__dataclass_fields__:
        cp = dataclasses.replace(cp, needs_layout_passes=False)
    f = pl.kernel(
        _edge_body,
        out_type=jax.ShapeDtypeStruct((NC, n, UW), F32),
        mesh=mesh,
        compiler_params=cp,
        scratch_types=[
            pltpu.VMEM((1, EC), jnp.int32),
            pltpu.VMEM((1, EC), jnp.int32),
            pltpu.VMEM((EC, 2 * H), F32),
            pltpu.VMEM((EC, 2 * H), F32),
            pltpu.VMEM((EC, UW), F32),
            pltpu.VMEM_SHARED((n, UW), F32),
            pltpu.SemaphoreType.DMA,
            pltpu.SemaphoreType.DMA,
        ],
    )
    return f(qs, kv, src, dst)


def _post_body(u_ref, s_ref, o_ref):
    u = u_ref[0] + u_ref[1]
    agg = u[:, 0:H]
    den = u[:, H:H + 1]
    o_ref[...] = jnp.maximum(agg / (den + 1e-16) + s_ref[:, H:2 * H], 0.0)


def _post(u, qs):
    n = qs.shape[0]
    rb = 2000
    return pl.pallas_call(
        _post_body,
        grid=(n // rb,),
        in_specs=[
            pl.BlockSpec((NC, rb, UW), lambda i: (0, i, 0)),
            pl.BlockSpec((rb, 2 * H), lambda i: (i, 0)),
        ],
        out_specs=pl.BlockSpec((rb, H), lambda i: (i, 0)),
        out_shape=jax.ShapeDtypeStruct((n, H), F32),
    )(u, qs)


def _pool_body(h_ref, b_ref, w1_ref, b1_ref, w2_ref, b2_ref, o_ref):
    g = o_ref.shape[0]
    n = h_ref.shape[0]
    iota = lax.broadcasted_iota(jnp.int32, (g, n), 0)
    onehot_t = (iota == b_ref[...]).astype(F32)
    sums = jnp.dot(onehot_t, h_ref[...], preferred_element_type=F32)
    cnt = jnp.sum(onehot_t, axis=1, keepdims=True)
    pooled = sums / jnp.maximum(cnt, 1.0)
    t = jnp.dot(pooled, w1_ref[...], preferred_element_type=F32) + b1_ref[...]
    o_ref[...] = jnp.dot(t, w2_ref[...], preferred_element_type=F32) + b2_ref[...]


def _pool(h, batch2d, g, w1, b1, w2, b2):
    return pl.pallas_call(
        _pool_body,
        out_shape=jax.ShapeDtypeStruct((g, 1), F32),
    )(h, batch2d, w1, b1, w2, b2)


def kernel(x, edge_index, batch,
           Wq1, bq1, Wk1, bk1, Wv1, bv1, Ws1, bs1,
           Wq2, bq2, Wk2, bk2, Wv2, bv2, Ws2, bs2,
           Wq3, bq3, Wk3, bk3, Wv3, bv3, Ws3, bs3,
           W_l1, b_l1, W_l2, b_l2):
    x = x.astype(F32)
    src = edge_index[0]
    dst = edge_index[1]
    layers = [
        (Wq1, bq1, Wk1, bk1, Wv1, bv1, Ws1, bs1),
        (Wq2, bq2, Wk2, bk2, Wv2, bv2, Ws2, bs2),
        (Wq3, bq3, Wk3, bk3, Wv3, bv3, Ws3, bs3),
    ]
    h = x
    for (Wq, bq, Wk, bk, Wv, bv, Ws, bs) in layers:
        wcat = jnp.concatenate([Wq, Ws, Wk, Wv], axis=1)
        bcat = jnp.concatenate([bq, bs, bk, bv]).reshape(1, -1)
        qs, kv = _qkvs(h, wcat, bcat)
        u = _edge_pass(qs, kv, src, dst)
        h = _post(u, qs)
    batch2d = batch.reshape(1, -1)
    return _pool(h, batch2d, 64,
                 W_l1, b_l1.reshape(1, -1), W_l2, b_l2.reshape(1, 1))
